# baseline (speedup 1.0000x reference)
import functools

import jax
import jax.numpy as jnp
from jax import lax
from jax.experimental import pallas as pl
from jax.experimental.pallas import tpu as pltpu
from jax.experimental.pallas import tpu_sc as plsc

VOCAB = 1000000
EMBED = 32
BATCH = 16384

SLAB = 128
LAST_TILE = (VOCAB // SLAB) * SLAB
LAST_W = VOCAB - LAST_TILE
MAX_RC = LAST_TILE - SLAB
RING = 16

_info = plsc.get_sparse_core_info()
_NC, _NS, _L = _info.num_cores, _info.num_subcores, _info.num_lanes
_NW = _NC * _NS
_B_PER_W = BATCH // _NW


def _make_kernel():
  mesh = plsc.VectorSubcoreMesh(core_axis_name="c", subcore_axis_name="s")

  @functools.partial(
      pl.kernel,
      mesh=mesh,
      out_type=jax.ShapeDtypeStruct((EMBED, BATCH), jnp.float32),
      compiler_params=pltpu.CompilerParams(
          needs_layout_passes=False,
          disable_bounds_checks=True,
          disable_semaphore_checks=True,
      ),
      scratch_types=[
          pltpu.VMEM((_B_PER_W + _L,), jnp.int32),
          pltpu.VMEM((RING, EMBED, SLAB), jnp.float32),
          pltpu.VMEM((EMBED, LAST_W), jnp.float32),
          pltpu.VMEM((EMBED, _B_PER_W), jnp.float32),
          pltpu.SemaphoreType.DMA((RING,)),
      ],
  )
  def gather_kernel(tT_hbm, idx_hbm, out_hbm, idx_v, slabs_v, last_v,
                    cols_v, sems):
    wid = lax.axis_index("s") * _NC + lax.axis_index("c")
    base = pl.multiple_of(wid * _B_PER_W, 128)
    pltpu.sync_copy(idx_hbm.at[pl.ds(base, _B_PER_W)],
                    idx_v.at[pl.ds(0, _B_PER_W)])
    pltpu.sync_copy(tT_hbm.at[:, pl.ds(LAST_TILE, LAST_W)], last_v)

    kvec0 = lax.iota(jnp.int32, _L)
    kvec1 = kvec0 + _L

    def read_idx(i):
      return idx_v[pl.ds(i, _L)][0]

    def rc_of(r):
      rc = lax.shift_left(lax.shift_right_logical(r, 7), 7)
      return pl.multiple_of(lax.min(rc, MAX_RC), 128)

    def issue(i, buf):
      r = read_idx(i)
      pltpu.async_copy(tT_hbm.at[:, pl.ds(rc_of(r), SLAB)],
                       slabs_v.at[buf], sems.at[buf])

    for j in range(RING):
      issue(j, j)

    def body(c, _):
      for j in range(RING):
        i = c * RING + j
        r = read_idx(i)
        pltpu.make_async_copy(tT_hbm.at[:, pl.ds(0, SLAB)],
                              slabs_v.at[j], sems.at[j]).wait()
        iv = jnp.full((_L,), 0, jnp.int32) + i

        @pl.when(r < LAST_TILE)
        def _():
          colv = jnp.full((_L,), 0, jnp.int32) + (r - rc_of(r))
          v0 = plsc.load_gather(slabs_v.at[j], [kvec0, colv])
          v1 = plsc.load_gather(slabs_v.at[j], [kvec1, colv])
          plsc.store_scatter(cols_v, [kvec0, iv], v0)
          plsc.store_scatter(cols_v, [kvec1, iv], v1)

        @pl.when(r >= LAST_TILE)
        def _():
          cv = jnp.full((_L,), 0, jnp.int32) + (r - LAST_TILE)
          v0 = plsc.load_gather(last_v, [kvec0, cv])
          v1 = plsc.load_gather(last_v, [kvec1, cv])
          plsc.store_scatter(cols_v, [kvec0, iv], v0)
          plsc.store_scatter(cols_v, [kvec1, iv], v1)

        nxt = i + RING

        @pl.when(nxt < _B_PER_W)
        def _():
          issue(nxt, j)
      return ()

    lax.fori_loop(0, _B_PER_W // RING, body, ())
    pltpu.sync_copy(cols_v, out_hbm.at[:, pl.ds(base, _B_PER_W)])

  return gather_kernel


_gather = _make_kernel()


@jax.jit
def kernel(data, table):
  outT = _gather(table.T, data.astype(jnp.int32))
  return outT.T


# speedup vs baseline: 1.0116x; 1.0116x over previous
import functools

import jax
import jax.numpy as jnp
from jax import lax
from jax.experimental import pallas as pl
from jax.experimental.pallas import tpu as pltpu
from jax.experimental.pallas import tpu_sc as plsc

VOCAB = 1000000
EMBED = 32
BATCH = 16384

SLAB = 128
LAST_TILE = (VOCAB // SLAB) * SLAB
LAST_W = VOCAB - LAST_TILE
MAX_RC = LAST_TILE - SLAB
RING = 8

_info = plsc.get_sparse_core_info()
_NC, _NS, _L = _info.num_cores, _info.num_subcores, _info.num_lanes
_NW = _NC * _NS
_B_PER_W = BATCH // _NW


def _make_kernel():
  mesh = plsc.VectorSubcoreMesh(core_axis_name="c", subcore_axis_name="s")

  @functools.partial(
      pl.kernel,
      mesh=mesh,
      out_type=jax.ShapeDtypeStruct((EMBED, BATCH), jnp.float32),
      compiler_params=pltpu.CompilerParams(needs_layout_passes=False),
      scratch_types=[
          pltpu.VMEM((_B_PER_W + _L,), jnp.int32),
          pltpu.VMEM((RING, EMBED, SLAB), jnp.float32),
          pltpu.VMEM((EMBED, LAST_W), jnp.float32),
          pltpu.VMEM((EMBED, _B_PER_W), jnp.float32),
          pltpu.SemaphoreType.DMA((RING,)),
      ],
  )
  def gather_kernel(tT_hbm, idx_hbm, out_hbm, idx_v, slabs_v, last_v,
                    cols_v, sems):
    wid = lax.axis_index("s") * _NC + lax.axis_index("c")
    base = pl.multiple_of(wid * _B_PER_W, 128)
    pltpu.sync_copy(idx_hbm.at[pl.ds(base, _B_PER_W)],
                    idx_v.at[pl.ds(0, _B_PER_W)])
    pltpu.sync_copy(tT_hbm.at[:, pl.ds(LAST_TILE, LAST_W)], last_v)

    kvec0 = lax.iota(jnp.int32, _L)
    kvec1 = kvec0 + _L

    def read_idx(i):
      return idx_v[pl.ds(i, _L)][0]

    def rc_of(r):
      rc = lax.shift_left(lax.shift_right_logical(r, 7), 7)
      return pl.multiple_of(lax.min(rc, MAX_RC), 128)

    def issue(i, buf):
      r = read_idx(i)
      pltpu.async_copy(tT_hbm.at[:, pl.ds(rc_of(r), SLAB)],
                       slabs_v.at[buf], sems.at[buf])

    for j in range(RING):
      issue(j, j)

    def body(c, _):
      for j in range(RING):
        i = c * RING + j
        r = read_idx(i)
        pltpu.make_async_copy(tT_hbm.at[:, pl.ds(0, SLAB)],
                              slabs_v.at[j], sems.at[j]).wait()
        iv = jnp.full((_L,), 0, jnp.int32) + i

        @pl.when(r < LAST_TILE)
        def _():
          colv = jnp.full((_L,), 0, jnp.int32) + (r - rc_of(r))
          v0 = plsc.load_gather(slabs_v.at[j], [kvec0, colv])
          v1 = plsc.load_gather(slabs_v.at[j], [kvec1, colv])
          plsc.store_scatter(cols_v, [kvec0, iv], v0)
          plsc.store_scatter(cols_v, [kvec1, iv], v1)

        @pl.when(r >= LAST_TILE)
        def _():
          cv = jnp.full((_L,), 0, jnp.int32) + (r - LAST_TILE)
          v0 = plsc.load_gather(last_v, [kvec0, cv])
          v1 = plsc.load_gather(last_v, [kvec1, cv])
          plsc.store_scatter(cols_v, [kvec0, iv], v0)
          plsc.store_scatter(cols_v, [kvec1, iv], v1)

        nxt = i + RING

        @pl.when(nxt < _B_PER_W)
        def _():
          issue(nxt, j)
      return ()

    lax.fori_loop(0, _B_PER_W // RING, body, ())
    pltpu.sync_copy(cols_v, out_hbm.at[:, pl.ds(base, _B_PER_W)])

  return gather_kernel


_gather = _make_kernel()


@jax.jit
def kernel(data, table):
  outT = _gather(table.T, data.astype(jnp.int32))
  return outT.T
